# baseline (speedup 1.0000x reference)
import functools

import jax
import jax.numpy as jnp
from jax import lax
from jax.experimental import pallas as pl
from jax.experimental.pallas import tpu as pltpu
from jax.experimental.pallas import tpu_sc as plsc

_THRESHOLD = jnp.float32(0.3)

_B, _D, _C = 16, 20000, 6
_ROWS = _B * _C
_NC, _NS = 2, 16
_NW = _NC * _NS
_L = 16


def _make_kernel():
    mesh = plsc.VectorSubcoreMesh(core_axis_name="c", subcore_axis_name="s")

    @functools.partial(
        pl.kernel,
        mesh=mesh,
        out_type=jax.ShapeDtypeStruct((_ROWS, _D), jnp.float32),
        scratch_types=[
            pltpu.VMEM((_D,), jnp.float32),
            pltpu.VMEM((_D,), jnp.float32),
            pltpu.VMEM((_D,), jnp.float32),
            pltpu.VMEM((_D,), jnp.float32),
            pltpu.SemaphoreType.DMA,
            pltpu.SemaphoreType.DMA,
        ],
        compiler_params=pltpu.CompilerParams(needs_layout_passes=False,
                                             use_tc_tiling_on_sc=True),
    )
    def _filter(x_hbm, out_hbm, cbuf, dbuf0, dbuf1, dbuf2, isem, osem):
        wid = lax.axis_index("s") * _NC + lax.axis_index("c")
        b = wid // 2
        h = wid % 2
        rows = [48 * h + 16 * j + b for j in range(3)]
        dbufs = [dbuf0, dbuf1, dbuf2]

        ins = [pltpu.make_async_copy(x_hbm.at[_B + b], cbuf, isem)]
        for j in range(3):
            ins.append(pltpu.make_async_copy(
                x_hbm.at[rows[j]], dbufs[j], isem))
        for cp in ins:
            cp.start()
        for cp in ins:
            cp.wait()

        def mask_vreg(i):
            cf = cbuf[pl.ds(i, _L)]
            keep = cf > _THRESHOLD
            zero = jnp.float32(0.0)
            for dbuf in dbufs:
                v = dbuf[pl.ds(i, _L)]
                dbuf[pl.ds(i, _L)] = jnp.where(keep, v, zero)

        plsc.parallel_loop(0, _D, _L, unroll=4)(mask_vreg)

        outs = [pltpu.make_async_copy(dbufs[j], out_hbm.at[rows[j]], osem)
                for j in range(3)]
        for cp in outs:
            cp.start()
        for cp in outs:
            cp.wait()

    return _filter


_FILTER = _make_kernel()


@jax.jit
def kernel(boxes):
    rows = boxes.transpose(2, 0, 1).reshape(_ROWS, _D)
    out = _FILTER(rows)
    return out.reshape(_C, _B, _D).transpose(1, 2, 0)


# speedup vs baseline: 1.0559x; 1.0559x over previous
import functools

import jax
import jax.numpy as jnp
from jax import lax
from jax.experimental import pallas as pl
from jax.experimental.pallas import tpu as pltpu
from jax.experimental.pallas import tpu_sc as plsc

_THRESHOLD = jnp.float32(0.3)

_B, _D, _C = 16, 20000, 6
_ROWS = _B * _C
_NC, _NS = 2, 16
_NW = _NC * _NS
_L = 16


def _make_kernel():
    mesh = plsc.VectorSubcoreMesh(core_axis_name="c", subcore_axis_name="s")

    @functools.partial(
        pl.kernel,
        mesh=mesh,
        out_type=jax.ShapeDtypeStruct((_ROWS, _D), jnp.float32),
        scratch_types=[
            pltpu.VMEM((_D,), jnp.float32),
            pltpu.VMEM((_D,), jnp.float32),
            pltpu.VMEM((_D,), jnp.float32),
            pltpu.VMEM((_D,), jnp.float32),
            pltpu.SemaphoreType.DMA,
            pltpu.SemaphoreType.DMA,
        ],
        compiler_params=pltpu.CompilerParams(needs_layout_passes=False,
                                             use_tc_tiling_on_sc=True),
    )
    def _filter(x_hbm, out_hbm, cbuf, dbuf0, dbuf1, dbuf2, isem, osem):
        wid = lax.axis_index("s") * _NC + lax.axis_index("c")
        b = wid // 2
        h = wid % 2
        rows = [48 * h + 16 * j + b for j in range(3)]
        dbufs = [dbuf0, dbuf1, dbuf2]

        ins = [pltpu.make_async_copy(x_hbm.at[_B + b], cbuf, isem)]
        for j in range(3):
            ins.append(pltpu.make_async_copy(
                x_hbm.at[rows[j]], dbufs[j], isem))
        for cp in ins:
            cp.start()
        ins[0].wait()

        def make_mask_row(dbuf):
            def mask_row(i):
                v = dbuf[pl.ds(i, _L)]
                cf = cbuf[pl.ds(i, _L)]
                dbuf[pl.ds(i, _L)] = jnp.where(cf > _THRESHOLD, v,
                                               jnp.float32(0.0))
            return mask_row

        outs = []
        for j in range(3):
            ins[j + 1].wait()
            plsc.parallel_loop(0, _D, _L, unroll=4)(make_mask_row(dbufs[j]))
            cp = pltpu.make_async_copy(dbufs[j], out_hbm.at[rows[j]], osem)
            cp.start()
            outs.append(cp)
        for cp in outs:
            cp.wait()

    return _filter


_FILTER = _make_kernel()


@jax.jit
def kernel(boxes):
    rows = boxes.transpose(2, 0, 1).reshape(_ROWS, _D)
    out = _FILTER(rows)
    return out.reshape(_C, _B, _D).transpose(1, 2, 0)


# speedup vs baseline: 1.0677x; 1.0112x over previous
import functools

import jax
import jax.numpy as jnp
from jax import lax
from jax.experimental import pallas as pl
from jax.experimental.pallas import tpu as pltpu
from jax.experimental.pallas import tpu_sc as plsc

_THRESHOLD = jnp.float32(0.3)

_B, _D, _C = 16, 20000, 6
_ROWS = _B * _C
_NC, _NS = 2, 16
_NW = _NC * _NS
_L = 16


def _make_kernel():
    mesh = plsc.VectorSubcoreMesh(core_axis_name="c", subcore_axis_name="s")

    @functools.partial(
        pl.kernel,
        mesh=mesh,
        out_type=jax.ShapeDtypeStruct((_ROWS, _D), jnp.float32),
        scratch_types=[
            pltpu.VMEM((_D,), jnp.float32),
            pltpu.VMEM((_D,), jnp.float32),
            pltpu.VMEM((_D,), jnp.float32),
            pltpu.VMEM((_D,), jnp.float32),
            pltpu.SemaphoreType.DMA,
            pltpu.SemaphoreType.DMA,
        ],
        compiler_params=pltpu.CompilerParams(needs_layout_passes=False,
                                             use_tc_tiling_on_sc=True),
    )
    def _filter(x_hbm, out_hbm, cbuf, dbuf0, dbuf1, dbuf2, isem, osem):
        wid = lax.axis_index("s") * _NC + lax.axis_index("c")
        b = wid // 2
        h = wid % 2
        rows = [48 * h + 16 * j + b for j in range(3)]
        dbufs = [dbuf0, dbuf1, dbuf2]

        ins = [pltpu.make_async_copy(x_hbm.at[_B + b], cbuf, isem)]
        for j in range(3):
            ins.append(pltpu.make_async_copy(
                x_hbm.at[rows[j]], dbufs[j], isem))
        for cp in ins:
            cp.start()
        ins[0].wait()

        def make_mask_row(dbuf):
            def mask_row(i):
                v = dbuf[pl.ds(i, _L)]
                cf = cbuf[pl.ds(i, _L)]
                dbuf[pl.ds(i, _L)] = jnp.where(cf > _THRESHOLD, v,
                                               jnp.float32(0.0))
            return mask_row

        outs = []
        for j in range(3):
            ins[j + 1].wait()
            plsc.parallel_loop(0, _D, _L, unroll=8)(make_mask_row(dbufs[j]))
            cp = pltpu.make_async_copy(dbufs[j], out_hbm.at[rows[j]], osem)
            cp.start()
            outs.append(cp)
        for cp in outs:
            cp.wait()

    return _filter


_FILTER = _make_kernel()


@jax.jit
def kernel(boxes):
    rows = boxes.transpose(2, 0, 1).reshape(_ROWS, _D)
    out = _FILTER(rows)
    return out.reshape(_C, _B, _D).transpose(1, 2, 0)
